# baseline (device time: 58900 ns/iter reference)
import jax
import jax.numpy as jnp
from jax import lax
from jax.experimental import pallas as pl
from jax.experimental.pallas import tpu as pltpu

N_DEV = 4
SIGMAS = (1, -1)


def _gelu(z):
    return 0.5 * z * (1.0 + jnp.tanh(0.7978845608 * (z + 0.044715 * z * z * z)))


def kernel(A, B):
    m, k_per = A.shape
    _, n = B.shape
    m_chunk = m // N_DEV
    n_half = n // 2

    def body(a_ref, b_ref, out_ref, a16, b16, partial_ref,
             rs1, rs2, st3, rs3, gsb, ag1, ag2, ag3, send_sems, recv_sems):
        my = lax.axis_index("i")
        left = lax.rem(my + N_DEV - 1, N_DEV)
        right = lax.rem(my + 1, N_DEV)

        def mod4(x):
            return lax.rem(x + 4 * N_DEV, N_DEV)

        def rows(c):
            return pl.ds(mod4(c) * m_chunk, m_chunk)

        def colh(sig):
            return pl.ds(0 if sig > 0 else n_half, n_half)

        def dev(sig):
            return right if sig > 0 else left

        def mm(c, sig):
            a_chunk = a16[rows(c), :]
            b_half = b16[:, 0:n_half] if sig > 0 else b16[:, n_half:n]
            partial_ref[rows(c), colh(sig)] = jnp.dot(
                a_chunk, b_half, preferred_element_type=jnp.float32
            ).astype(jnp.bfloat16)

        a16[:, :] = a_ref[:, :].astype(jnp.bfloat16)
        b16[:, :] = b_ref[:, :].astype(jnp.bfloat16)

        barrier_sem = pltpu.get_barrier_semaphore()
        for nbr in (left, right):
            pl.semaphore_signal(
                barrier_sem, inc=1,
                device_id=(nbr,), device_id_type=pl.DeviceIdType.MESH,
            )
        pl.semaphore_wait(barrier_sem, 2)

        def mk(f, si, src, dst, sig_dir):
            i = f * 2 + si
            return pltpu.make_async_remote_copy(
                src_ref=src, dst_ref=dst,
                send_sem=send_sems.at[i], recv_sem=recv_sems.at[i],
                device_id=(dev(sig_dir),),
                device_id_type=pl.DeviceIdType.MESH,
            )

        F1, F2, F3, F4, F5, F6 = {}, {}, {}, {}, {}, {}
        for si, s in enumerate(SIGMAS):
            F1[s] = mk(0, si, partial_ref.at[rows(my + s), colh(s)],
                       rs1.at[si], s)
            F2[s] = mk(1, si, partial_ref.at[rows(my - 2 * s), colh(s)],
                       rs2.at[si], -s)
            F3[s] = mk(2, si, st3.at[si], rs3.at[si], -s)
            F4[s] = mk(3, si, gsb.at[si], ag1.at[si], s)
            F5[s] = mk(4, si, gsb.at[si], ag2.at[si], -s)
            F6[s] = mk(5, si, ag1.at[si], ag3.at[si], s)

        mm(my + 1, 1)
        F1[1].start()
        mm(my - 1, -1)
        F1[-1].start()
        mm(my - 2, 1)
        F2[1].start()
        mm(my + 2, -1)
        F2[-1].start()
        mm(my - 1, 1)
        mm(my + 1, -1)
        mm(my, 1)
        mm(my, -1)

        for si, s in enumerate(SIGMAS):
            F2[s].wait_recv()
            st3[si, :, :] = rs2[si, :, :] + partial_ref[rows(my - s), colh(s)]
            F3[s].start()

        for si, s in enumerate(SIGMAS):
            F1[s].wait_recv()
            F3[s].wait_recv()
            g = _gelu(partial_ref[rows(my), colh(s)].astype(jnp.float32)
                      + rs1[si, :, :].astype(jnp.float32)
                      + rs3[si, :, :].astype(jnp.float32))
            gsb[si, :, :] = g.astype(jnp.bfloat16)
            F4[s].start()
            F5[s].start()
            out_ref[rows(my), colh(s)] = g

        for si, s in enumerate(SIGMAS):
            F4[s].wait_recv()
            F6[s].start()
            out_ref[rows(my - s), colh(s)] = ag1[si, :, :].astype(jnp.float32)
        for si, s in enumerate(SIGMAS):
            F5[s].wait_recv()
            out_ref[rows(my + s), colh(s)] = ag2[si, :, :].astype(jnp.float32)
        for si, s in enumerate(SIGMAS):
            F6[s].wait_recv()
            out_ref[rows(my - 2 * s), colh(s)] = ag3[si, :, :].astype(jnp.float32)

        for d in (F1, F2, F3, F4, F5, F6):
            for s in SIGMAS:
                d[s].wait_send()

    buf = (2, m_chunk, n_half)
    return pl.pallas_call(
        body,
        out_shape=jax.ShapeDtypeStruct((m, n), jnp.float32),
        in_specs=[
            pl.BlockSpec(memory_space=pltpu.VMEM),
            pl.BlockSpec(memory_space=pltpu.VMEM),
        ],
        out_specs=pl.BlockSpec(memory_space=pltpu.VMEM),
        scratch_shapes=[
            pltpu.VMEM((m, k_per), jnp.bfloat16),
            pltpu.VMEM((k_per, n), jnp.bfloat16),
            pltpu.VMEM((m, n), jnp.bfloat16),
            pltpu.VMEM(buf, jnp.bfloat16),
            pltpu.VMEM(buf, jnp.bfloat16),
            pltpu.VMEM(buf, jnp.bfloat16),
            pltpu.VMEM(buf, jnp.bfloat16),
            pltpu.VMEM(buf, jnp.bfloat16),
            pltpu.VMEM(buf, jnp.bfloat16),
            pltpu.VMEM(buf, jnp.bfloat16),
            pltpu.VMEM(buf, jnp.bfloat16),
            pltpu.SemaphoreType.DMA((12,)),
            pltpu.SemaphoreType.DMA((12,)),
        ],
        compiler_params=pltpu.CompilerParams(collective_id=0),
    )(A, B)


# device time: 53460 ns/iter; 1.1018x vs baseline; 1.1018x over previous
import jax
import jax.numpy as jnp
from jax import lax
from jax.experimental import pallas as pl
from jax.experimental.pallas import tpu as pltpu

N_DEV = 4
N_STEP = N_DEV - 1
N_LANE = 4


def _gelu(z):
    return 0.5 * z * (1.0 + jnp.tanh(0.7978845608 * (z + 0.044715 * z * z * z)))


def kernel(A, B):
    m, k_per = A.shape
    _, n = B.shape
    m_chunk = m // N_DEV
    n_q = n // N_LANE

    def body(a_ref, b_ref, out_ref, a16, b16, partial_ref,
             srs, rsb, gsb, agb, rs_send, rs_recv, ag_send, ag_recv):
        my = lax.axis_index("i")
        left = lax.rem(my + N_DEV - 1, N_DEV)
        right = lax.rem(my + 1, N_DEV)

        def mod4(x):
            return lax.rem(x + 2 * N_DEV, N_DEV)

        def rows(c):
            return pl.ds(c * m_chunk, m_chunk)

        lanes = [(0, 1), (2, -1), (1, 1), (3, -1)]

        def cq(lane):
            return pl.ds(lanes[lane][0] * n_q, n_q)

        def dev(lane):
            return right if lanes[lane][1] > 0 else left

        def c_recv(lane, s):
            return mod4(my - lanes[lane][1] * (s + 1))

        def mm(c, col_lo):
            a_chunk = a16[rows(c), :]
            b_half = b16[:, 0:2 * n_q] if col_lo else b16[:, 2 * n_q:n]
            partial_ref[rows(c), pl.ds(0, 2 * n_q) if col_lo
                        else pl.ds(2 * n_q, 2 * n_q)] = jnp.dot(
                a_chunk, b_half, preferred_element_type=jnp.float32
            ).astype(jnp.bfloat16)

        a16[:, :] = a_ref[:, :].astype(jnp.bfloat16)
        b16[:, :] = b_ref[:, :].astype(jnp.bfloat16)

        barrier_sem = pltpu.get_barrier_semaphore()
        for nbr in (left, right):
            pl.semaphore_signal(
                barrier_sem, inc=1,
                device_id=(nbr,), device_id_type=pl.DeviceIdType.MESH,
            )
        pl.semaphore_wait(barrier_sem, 2)

        def sem_i(lane, s):
            return lane * N_STEP + s

        rs_d = [[pltpu.make_async_remote_copy(
                    src_ref=srs.at[k, s], dst_ref=rsb.at[k, s],
                    send_sem=rs_send.at[sem_i(k, s)],
                    recv_sem=rs_recv.at[sem_i(k, s)],
                    device_id=(dev(k),), device_id_type=pl.DeviceIdType.MESH)
                 for s in range(N_STEP)] for k in range(N_LANE)]
        ag_d = [[pltpu.make_async_remote_copy(
                    src_ref=(gsb.at[k] if h == 0 else agb.at[k, h - 1]),
                    dst_ref=agb.at[k, h],
                    send_sem=ag_send.at[sem_i(k, h)],
                    recv_sem=ag_recv.at[sem_i(k, h)],
                    device_id=(dev(k),), device_id_type=pl.DeviceIdType.MESH)
                 for h in range(N_STEP)] for k in range(N_LANE)]

        mm(my, True)
        mm(my, False)
        for k in range(N_LANE):
            srs[k, 0, :, :] = partial_ref[rows(my), cq(k)]
            rs_d[k][0].start()
        mm(mod4(my - 1), True)
        mm(mod4(my + 1), False)
        mm(mod4(my - 2), True)
        mm(mod4(my + 2), False)
        mm(mod4(my + 1), True)
        mm(mod4(my - 1), False)

        for s in range(N_STEP):
            for k in range(N_LANE):
                rs_d[k][s].wait()
                c = c_recv(k, s)
                if s < N_STEP - 1:
                    srs[k, s + 1, :, :] = (
                        rsb[k, s, :, :] + partial_ref[rows(c), cq(k)]
                    )
                    rs_d[k][s + 1].start()
                else:
                    g = _gelu(rsb[k, s, :, :].astype(jnp.float32)
                              + partial_ref[rows(c), cq(k)].astype(jnp.float32))
                    gsb[k, :, :] = g.astype(jnp.bfloat16)
                    ag_d[k][0].start()
                    out_ref[rows(mod4(my + lanes[k][1])), cq(k)] = g

        for h in range(N_STEP):
            for k in range(N_LANE):
                ag_d[k][h].wait_recv()
                if h < N_STEP - 1:
                    ag_d[k][h + 1].start()
            for k in range(N_LANE):
                out_ref[rows(mod4(my - lanes[k][1] * h)), cq(k)] = (
                    agb[k, h, :, :].astype(jnp.float32)
                )
        for k in range(N_LANE):
            for h in range(N_STEP):
                ag_d[k][h].wait_send()

    lane_shape = (N_LANE, N_STEP, m_chunk, n_q)
    n_sem = N_LANE * N_STEP
    return pl.pallas_call(
        body,
        out_shape=jax.ShapeDtypeStruct((m, n), jnp.float32),
        in_specs=[
            pl.BlockSpec(memory_space=pltpu.VMEM),
            pl.BlockSpec(memory_space=pltpu.VMEM),
        ],
        out_specs=pl.BlockSpec(memory_space=pltpu.VMEM),
        scratch_shapes=[
            pltpu.VMEM((m, k_per), jnp.bfloat16),
            pltpu.VMEM((k_per, n), jnp.bfloat16),
            pltpu.VMEM((m, n), jnp.bfloat16),
            pltpu.VMEM(lane_shape, jnp.bfloat16),
            pltpu.VMEM(lane_shape, jnp.bfloat16),
            pltpu.VMEM((N_LANE, m_chunk, n_q), jnp.bfloat16),
            pltpu.VMEM(lane_shape, jnp.bfloat16),
            pltpu.SemaphoreType.DMA((n_sem,)),
            pltpu.SemaphoreType.DMA((n_sem,)),
            pltpu.SemaphoreType.DMA((n_sem,)),
            pltpu.SemaphoreType.DMA((n_sem,)),
        ],
        compiler_params=pltpu.CompilerParams(collective_id=0),
    )(A, B)
